# baseline (speedup 1.0000x reference)
import functools

import jax
import jax.numpy as jnp
from jax import lax
from jax.experimental import pallas as pl
from jax.experimental import layout as jex_layout
from jax.experimental.pallas import tpu as pltpu
from jax.experimental.pallas import tpu_sc as plsc

NC = 2
NS = 16
NW = NC * NS
NBUF = 4


@functools.partial(jax.jit, static_argnames=("h", "v", "d"))
def _embed_sc(xt, weight, h, v, d):
    b = xt.shape[1]
    bw = b // NW

    mesh = plsc.VectorSubcoreMesh(core_axis_name="c", subcore_axis_name="s")

    @functools.partial(
        pl.kernel,
        out_type=jax.ShapeDtypeStruct((h, d, b), jnp.float32),
        mesh=mesh,
        compiler_params=pltpu.CompilerParams(
            use_tc_tiling_on_sc=False, needs_layout_passes=False
        ),
        scratch_types=[
            pltpu.VMEM((h, bw), jnp.int32),
            *([pltpu.VMEM((bw, d), jnp.float32)] * NBUF),
            *([pltpu.VMEM((d, bw), jnp.float32)] * NBUF),
            *([pltpu.SemaphoreType.DMA] * NBUF),
            *([pltpu.SemaphoreType.DMA] * NBUF),
        ],
    )
    def body(xt_hbm, table_hbm, out_hbm, idx_v, *rest):
        gbuf = rest[:NBUF]
        tbuf = rest[NBUF : 2 * NBUF]
        gsem = rest[2 * NBUF : 3 * NBUF]
        wsem = rest[3 * NBUF : 4 * NBUF]
        wid = lax.axis_index("s") * NC + lax.axis_index("c")
        b0 = wid * bw
        pltpu.sync_copy(xt_hbm.at[:, pl.ds(b0, bw)], idx_v)

        iota = lax.iota(jnp.int32, 16)
        rvecs = [v16 * 16 + iota for v16 in range(bw // 16)]

        for s in range(NBUF):
            pltpu.async_copy(table_hbm.at[idx_v.at[s]], gbuf[s], gsem[s])

        def block(g, _):
            for s in range(NBUF):
                hh = g * NBUF + s
                pltpu.make_async_copy(
                    table_hbm.at[idx_v.at[0]], gbuf[s], gsem[s]
                ).wait()

                @pl.when(g > 0)
                def _():
                    pltpu.make_async_copy(
                        tbuf[s], out_hbm.at[0, :, pl.ds(b0, bw)], wsem[s]
                    ).wait()

                for c in range(d):
                    cvec = jnp.full((16,), c, jnp.int32)
                    vals = [
                        plsc.load_gather(gbuf[s], [rvecs[v16], cvec])
                        for v16 in range(bw // 16)
                    ]
                    for v16 in range(bw // 16):
                        tbuf[s][c, pl.ds(v16 * 16, 16)] = vals[v16]

                pltpu.async_copy(
                    tbuf[s], out_hbm.at[hh, :, pl.ds(b0, bw)], wsem[s]
                )

                @pl.when(g < h // NBUF - 1)
                def _():
                    pltpu.async_copy(
                        table_hbm.at[idx_v.at[hh + NBUF]], gbuf[s], gsem[s]
                    )

            return _

        lax.fori_loop(0, h // NBUF, block, None)
        for s in range(NBUF):
            pltpu.make_async_copy(
                tbuf[s], out_hbm.at[0, :, pl.ds(b0, bw)], wsem[s]
            ).wait()

    return body(xt, weight)


def kernel(x, weight):
    b, h = x.shape
    v, d = weight.shape
    xt = x.T.astype(jnp.int32)
    wt = jex_layout.with_layout_constraint(
        weight, jex_layout.Layout((1, 0), tiling=((8,),))
    )
    out3 = _embed_sc(xt, wt, h, v, d)
    return out3.transpose(2, 0, 1)


# speedup vs baseline: 1.7826x; 1.7826x over previous
import functools

import jax
import jax.numpy as jnp
from jax import lax
from jax.experimental import pallas as pl
from jax.experimental import layout as jex_layout
from jax.experimental.pallas import tpu as pltpu
from jax.experimental.pallas import tpu_sc as plsc

NC = 2
NS = 16
NW = NC * NS
NBUF = 2


@functools.partial(jax.jit, static_argnames=("h", "v", "d"))
def _embed_sc(xt, weight, h, v, d):
    b = xt.shape[1]
    bw = b // NW

    mesh = plsc.VectorSubcoreMesh(core_axis_name="c", subcore_axis_name="s")

    @functools.partial(
        pl.kernel,
        out_type=jax.ShapeDtypeStruct((h, d, b), jnp.float32),
        mesh=mesh,
        compiler_params=pltpu.CompilerParams(
            use_tc_tiling_on_sc=False, needs_layout_passes=False
        ),
        scratch_types=[
            pltpu.VMEM((h, bw), jnp.int32),
            *([pltpu.VMEM((bw, d), jnp.float32)] * NBUF),
            *([pltpu.VMEM((d, bw), jnp.float32)] * NBUF),
            *([pltpu.SemaphoreType.DMA] * NBUF),
            *([pltpu.SemaphoreType.DMA] * NBUF),
        ],
    )
    def body(xt_hbm, table_hbm, out_hbm, idx_v, *rest):
        gbuf = rest[:NBUF]
        tbuf = rest[NBUF : 2 * NBUF]
        gsem = rest[2 * NBUF : 3 * NBUF]
        wsem = rest[3 * NBUF : 4 * NBUF]
        wid = lax.axis_index("s") * NC + lax.axis_index("c")
        b0 = wid * bw
        pltpu.sync_copy(xt_hbm.at[:, pl.ds(b0, bw)], idx_v)

        iota = lax.iota(jnp.int32, 16)
        rvecs = [v16 * 16 + iota for v16 in range(bw // 16)]

        for s in range(NBUF):
            pltpu.async_copy(table_hbm.at[idx_v.at[s]], gbuf[s], gsem[s])

        def block(g, _):
            for s in range(NBUF):
                hh = g * NBUF + s
                pltpu.make_async_copy(
                    table_hbm.at[idx_v.at[0]], gbuf[s], gsem[s]
                ).wait()

                @pl.when(g > 0)
                def _():
                    pltpu.make_async_copy(
                        tbuf[s], out_hbm.at[0, :, pl.ds(b0, bw)], wsem[s]
                    ).wait()

                def cb_body(cb, carry):
                    coff = cb * 16
                    for dg in range(16):
                        cdiag = ((iota + dg) & 15) + coff
                        vals = [
                            plsc.load_gather(gbuf[s], [rvecs[v16], cdiag])
                            for v16 in range(bw // 16)
                        ]
                        for v16 in range(bw // 16):
                            plsc.store_scatter(
                                tbuf[s], [cdiag, rvecs[v16]], vals[v16]
                            )
                    return carry

                lax.fori_loop(0, d // 16, cb_body, None)

                pltpu.async_copy(
                    tbuf[s], out_hbm.at[hh, :, pl.ds(b0, bw)], wsem[s]
                )

                @pl.when(g < h // NBUF - 1)
                def _():
                    pltpu.async_copy(
                        table_hbm.at[idx_v.at[hh + NBUF]], gbuf[s], gsem[s]
                    )

            return _

        lax.fori_loop(0, h // NBUF, block, None)
        for s in range(NBUF):
            pltpu.make_async_copy(
                tbuf[s], out_hbm.at[0, :, pl.ds(b0, bw)], wsem[s]
            ).wait()

    return body(xt, weight)


def kernel(x, weight):
    b, h = x.shape
    v, d = weight.shape
    xt = x.T.astype(jnp.int32)
    wt = jex_layout.with_layout_constraint(
        weight, jex_layout.Layout((1, 0), tiling=((8,),))
    )
    out3 = _embed_sc(xt, wt, h, v, d)
    return out3.transpose(2, 0, 1)


# speedup vs baseline: 1.7844x; 1.0010x over previous
import functools

import jax
import jax.numpy as jnp
from jax import lax
from jax.experimental import pallas as pl
from jax.experimental.pallas import tpu as pltpu
from jax.experimental.pallas import tpu_sc as plsc

NC = 2
NS = 16
NW = NC * NS
NBUF = 2


@functools.partial(jax.jit, static_argnames=("h", "v", "d"))
def _embed_sc(xt, weight, h, v, d):
    b = xt.shape[1]
    bw = b // NW

    mesh = plsc.VectorSubcoreMesh(core_axis_name="c", subcore_axis_name="s")

    @functools.partial(
        pl.kernel,
        out_type=jax.ShapeDtypeStruct((h, d, b), jnp.float32),
        mesh=mesh,
        compiler_params=pltpu.CompilerParams(
            use_tc_tiling_on_sc=False, needs_layout_passes=False
        ),
        scratch_types=[
            pltpu.VMEM((h, bw), jnp.int32),
            *([pltpu.VMEM((bw, d), jnp.float32)] * NBUF),
            *([pltpu.VMEM((d, bw), jnp.float32)] * NBUF),
            *([pltpu.SemaphoreType.DMA] * NBUF),
            *([pltpu.SemaphoreType.DMA] * NBUF),
        ],
    )
    def body(xt_hbm, table_hbm, out_hbm, idx_v, *rest):
        gbuf = rest[:NBUF]
        tbuf = rest[NBUF : 2 * NBUF]
        gsem = rest[2 * NBUF : 3 * NBUF]
        wsem = rest[3 * NBUF : 4 * NBUF]
        wid = lax.axis_index("s") * NC + lax.axis_index("c")
        b0 = wid * bw
        pltpu.sync_copy(xt_hbm.at[:, pl.ds(b0, bw)], idx_v)

        iota = lax.iota(jnp.int32, 16)
        rvecs = [v16 * 16 + iota for v16 in range(bw // 16)]

        for s in range(NBUF):
            pltpu.async_copy(table_hbm.at[idx_v.at[s]], gbuf[s], gsem[s])

        def block(g, _):
            for s in range(NBUF):
                hh = g * NBUF + s
                pltpu.make_async_copy(
                    table_hbm.at[idx_v.at[0]], gbuf[s], gsem[s]
                ).wait()

                @pl.when(g > 0)
                def _():
                    pltpu.make_async_copy(
                        tbuf[s], out_hbm.at[0, :, pl.ds(b0, bw)], wsem[s]
                    ).wait()

                def cb_body(cb, carry):
                    coff = cb * 16
                    for dg in range(16):
                        cdiag = ((iota + dg) & 15) + coff
                        vals = [
                            plsc.load_gather(gbuf[s], [rvecs[v16], cdiag])
                            for v16 in range(bw // 16)
                        ]
                        for v16 in range(bw // 16):
                            plsc.store_scatter(
                                tbuf[s], [cdiag, rvecs[v16]], vals[v16]
                            )
                    return carry

                lax.fori_loop(0, d // 16, cb_body, None)

                pltpu.async_copy(
                    tbuf[s], out_hbm.at[hh, :, pl.ds(b0, bw)], wsem[s]
                )

                @pl.when(g < h // NBUF - 1)
                def _():
                    pltpu.async_copy(
                        table_hbm.at[idx_v.at[hh + NBUF]], gbuf[s], gsem[s]
                    )

            return _

        lax.fori_loop(0, h // NBUF, block, None)
        for s in range(NBUF):
            pltpu.make_async_copy(
                tbuf[s], out_hbm.at[0, :, pl.ds(b0, bw)], wsem[s]
            ).wait()

    return body(xt, weight)


def kernel(x, weight):
    b, h = x.shape
    v, d = weight.shape
    xt = x.T.astype(jnp.int32)
    out3 = _embed_sc(xt, weight, h, v, d)
    return out3.transpose(2, 0, 1)
